# baseline (speedup 1.0000x reference)
import functools

import jax
import jax.numpy as jnp
from jax import lax
from jax.experimental import pallas as pl
from jax.experimental.pallas import tpu as pltpu
from jax.experimental.pallas import tpu_sc as plsc

_NC = 2
_NS = 16
_NW = _NC * _NS
_LANES = 16


def _tc_pack_tail(weights, *, CH, CTP):
    N, C = weights.shape
    CT = C - CH
    BLK = 512
    grid = (N // BLK,)

    def body(w_ref, o_ref):
        o_ref[...] = jnp.concatenate(
            [w_ref[:, CH:C], jnp.zeros((BLK, CTP - CT), jnp.float32)], axis=1)

    return pl.pallas_call(
        body,
        grid=grid,
        in_specs=[pl.BlockSpec((BLK, C), lambda j: (j, 0))],
        out_specs=pl.BlockSpec((BLK, CTP), lambda j: (j, 0)),
        out_shape=jax.ShapeDtypeStruct((N, CTP), jnp.float32),
    )(weights)


def _sc_stats(output, idx3, weights, wtail, *, B, C, CH, bpw, K, nchunk):
    hf = CH // _LANES
    CT = C - CH
    tf = CT // _LANES
    trem = CT - tf * _LANES

    mesh = plsc.VectorSubcoreMesh(core_axis_name="c", subcore_axis_name="s")

    @functools.partial(
        pl.kernel,
        mesh=mesh,
        out_type=[
            jax.ShapeDtypeStruct((B, _LANES), jnp.float32),
            jax.ShapeDtypeStruct((_NW, _LANES), jnp.float32),
        ],
        scratch_types=[
            pltpu.VMEM((nchunk, K), jnp.int32),
            pltpu.VMEM((K, CH), jnp.float32),
            pltpu.VMEM((K, CH), jnp.float32),
            pltpu.VMEM((K, wtail.shape[1]), jnp.float32),
            pltpu.VMEM((K, wtail.shape[1]), jnp.float32),
            pltpu.VMEM((K, C), jnp.float32),
            pltpu.VMEM((K, C), jnp.float32),
            pltpu.VMEM((bpw, _LANES), jnp.float32),
            pltpu.VMEM((_LANES,), jnp.float32),
            pltpu.SemaphoreType.DMA,
            pltpu.SemaphoreType.DMA,
        ],
    )
    def k(out_hbm, idx_hbm, w_hbm, wt_hbm, g2_hbm, t1_hbm,
          idx_v, w0_v, w1_v, x0_v, x1_v, o0_v, o1_v, g2_v, t1_v, sem0, sem1):
        cid = lax.axis_index("c")
        sid = lax.axis_index("s")
        wid = sid * _NC + cid
        base = wid * bpw

        pltpu.sync_copy(idx_hbm.at[wid], idx_v)

        w_bufs, x_bufs = (w0_v, w1_v), (x0_v, x1_v)
        o_bufs, sems = (o0_v, o1_v), (sem0, sem1)

        def copies(ch):
            p = ch % 2
            return (
                pltpu.make_async_copy(
                    w_hbm.at[idx_v.at[ch], pl.ds(0, CH)], w_bufs[p], sems[p]),
                pltpu.make_async_copy(
                    wt_hbm.at[idx_v.at[ch]], x_bufs[p], sems[p]),
                pltpu.make_async_copy(
                    out_hbm.at[pl.ds(base + ch * K, K)], o_bufs[p], sems[p]),
            )

        for c in copies(0):
            c.start()

        tailmask = jnp.where(lax.iota(jnp.int32, _LANES) < (_LANES - trem),
                             0.0, 1.0).astype(jnp.float32)

        acc1 = jnp.zeros((_LANES,), jnp.float32)
        for ch in range(nchunk):
            if ch + 1 < nchunk:
                for c in copies(ch + 1):
                    c.start()
            for c in copies(ch):
                c.wait()
            w_v, x_v, o_v = w_bufs[ch % 2], x_bufs[ch % 2], o_bufs[ch % 2]

            def row_body(r, a1):
                def head_body(j, carry):
                    c1, c2 = carry
                    off = pl.multiple_of(j * _LANES, _LANES)
                    wv = w_v[r, pl.ds(off, _LANES)]
                    ov = o_v[r, pl.ds(off, _LANES)]
                    return c1 + wv * ov, c2 + wv

                def tail_body(j, carry):
                    c1, c2 = carry
                    off = pl.multiple_of(j * _LANES, _LANES)
                    off2 = pl.multiple_of(CH + j * _LANES, _LANES)
                    wv = x_v[r, pl.ds(off, _LANES)]
                    ov = o_v[r, pl.ds(off2, _LANES)]
                    return c1 + wv * ov, c2 + wv

                carry = lax.fori_loop(
                    0, hf, head_body,
                    (a1, jnp.zeros((_LANES,), jnp.float32)))
                c1, c2 = lax.fori_loop(0, tf, tail_body, carry)
                wv = x_v[r, pl.ds(CT - _LANES, _LANES)] * tailmask
                ov = o_v[r, pl.ds(C - _LANES, _LANES)]
                c1 = c1 + wv * ov
                c2 = c2 + wv
                g2_v[ch * K + r, :] = c2
                return c1

            acc1 = lax.fori_loop(0, K, row_body, acc1)

        t1_v[:] = acc1
        pltpu.sync_copy(g2_v, g2_hbm.at[pl.ds(base, bpw)])
        pltpu.sync_copy(t1_v, t1_hbm.at[wid])

    return k(output, idx3, weights, wtail)


def _tc_lse(output, *, B, C):
    BLK = 256
    grid = (B // BLK,)

    def body(out_ref, lse_ref):
        x = out_ref[...]
        m = jnp.max(x, axis=1, keepdims=True)
        lse_ref[...] = m + jnp.log(jnp.sum(jnp.exp(x - m), axis=1,
                                           keepdims=True))

    return pl.pallas_call(
        body,
        grid=grid,
        in_specs=[pl.BlockSpec((BLK, C), lambda j: (j, 0))],
        out_specs=pl.BlockSpec((BLK, 1), lambda j: (j, 0)),
        out_shape=jax.ShapeDtypeStruct((B, 1), jnp.float32),
    )(output)


def _tc_combine(lse, g2part, t1part, *, B):
    def body(lse_ref, g2_ref, t1_ref, L_ref):
        g2 = jnp.sum(g2_ref[...], axis=1, keepdims=True)
        L_ref[...] = (
            jnp.sum(lse_ref[...] * g2, keepdims=True).reshape(1, 1)
            - jnp.sum(t1_ref[...], keepdims=True).reshape(1, 1)
        ) / B

    L = pl.pallas_call(
        body,
        out_shape=jax.ShapeDtypeStruct((1, 1), jnp.float32),
    )(lse, g2part, t1part)
    return L[0, 0]


def kernel(output, targets, indices, weights):
    B, C = output.shape
    CH = C // 128 * 128
    bpw = B // _NW
    K = 16
    nchunk = bpw // K
    idx3 = indices.reshape(_NW, nchunk, K)
    wtail = _tc_pack_tail(weights, CH=CH, CTP=128)
    g2part, t1part = _sc_stats(
        output, idx3, weights, wtail,
        B=B, C=C, CH=CH, bpw=bpw, K=K, nchunk=nchunk)
    lse = _tc_lse(output, B=B, C=C)
    return _tc_combine(lse, g2part, t1part, B=B)


# speedup vs baseline: 1.3560x; 1.3560x over previous
import functools

import jax
import jax.numpy as jnp
from jax import lax
from jax.experimental import pallas as pl
from jax.experimental.pallas import tpu as pltpu
from jax.experimental.pallas import tpu_sc as plsc

_NC = 2
_NS = 16
_NW = _NC * _NS
_LANES = 16
_UNROLL = 8


def _sc_head_stats(output, idx3, weights, *, B, CH, bpw, K, nchunk):
    hf = CH // _LANES
    ho = hf // _UNROLL

    mesh = plsc.VectorSubcoreMesh(core_axis_name="c", subcore_axis_name="s")

    @functools.partial(
        pl.kernel,
        mesh=mesh,
        out_type=[
            jax.ShapeDtypeStruct((B, _LANES), jnp.float32),
            jax.ShapeDtypeStruct((_NW, _LANES), jnp.float32),
        ],
        scratch_types=[
            pltpu.VMEM((nchunk, K), jnp.int32),
            pltpu.VMEM((K, CH), jnp.float32),
            pltpu.VMEM((K, CH), jnp.float32),
            pltpu.VMEM((K, CH), jnp.float32),
            pltpu.VMEM((K, CH), jnp.float32),
            pltpu.VMEM((bpw, _LANES), jnp.float32),
            pltpu.VMEM((_LANES,), jnp.float32),
            pltpu.SemaphoreType.DMA,
            pltpu.SemaphoreType.DMA,
        ],
    )
    def k(out_hbm, idx_hbm, w_hbm, g2_hbm, t1_hbm,
          idx_v, w0_v, w1_v, o0_v, o1_v, g2_v, t1_v, sem0, sem1):
        cid = lax.axis_index("c")
        sid = lax.axis_index("s")
        wid = sid * _NC + cid
        base = wid * bpw

        pltpu.sync_copy(idx_hbm.at[wid], idx_v)

        w_bufs, o_bufs, sems = (w0_v, w1_v), (o0_v, o1_v), (sem0, sem1)

        def copies(ch):
            p = ch % 2
            return (
                pltpu.make_async_copy(
                    w_hbm.at[idx_v.at[ch], pl.ds(0, CH)], w_bufs[p], sems[p]),
                pltpu.make_async_copy(
                    out_hbm.at[pl.ds(base + ch * K, K), pl.ds(0, CH)],
                    o_bufs[p], sems[p]),
            )

        for c in copies(0):
            c.start()

        acc1 = jnp.zeros((_LANES,), jnp.float32)
        for ch in range(nchunk):
            if ch + 1 < nchunk:
                for c in copies(ch + 1):
                    c.start()
            for c in copies(ch):
                c.wait()
            w_v, o_v = w_bufs[ch % 2], o_bufs[ch % 2]

            def row_body(r, a1):
                def head_body(j, carry):
                    c1, c2 = carry
                    jbase = pl.multiple_of(j * (_LANES * _UNROLL),
                                           _LANES * _UNROLL)
                    for u in range(_UNROLL):
                        wv = w_v[r, pl.ds(jbase + u * _LANES, _LANES)]
                        ov = o_v[r, pl.ds(jbase + u * _LANES, _LANES)]
                        c1 = c1 + wv * ov
                        c2 = c2 + wv
                    return c1, c2

                a1, a2 = lax.fori_loop(
                    0, ho, head_body, (a1, jnp.zeros((_LANES,), jnp.float32)))
                g2_v[ch * K + r, :] = a2
                return a1

            acc1 = lax.fori_loop(0, K, row_body, acc1)

        t1_v[:] = acc1
        pltpu.sync_copy(g2_v, g2_hbm.at[pl.ds(base, bpw)])
        pltpu.sync_copy(t1_v, t1_hbm.at[wid])

    return k(output, idx3, weights)


def _tc_lse_tail(output, idx2, weights, *, B, C, CH):
    BLK = 128
    CT = C - CH
    G = B // BLK
    grid = (G,)

    def body(idx_ref, out_ref, w_hbm, lse_ref, s_ref, t0, t1, sem0, sem1):
        j = pl.program_id(0)
        bufs, sems = (t0, t1), (sem0, sem1)

        def start_tails(step, p):
            for r in range(BLK):
                pltpu.make_async_copy(
                    w_hbm.at[pl.ds(idx_ref[step, 0, r], 1), pl.ds(CH, CT)],
                    bufs[p].at[pl.ds(r, 1), :],
                    sems[p],
                ).start()

        def wait_tails(p):
            for r in range(BLK):
                pltpu.make_async_copy(
                    w_hbm.at[pl.ds(0, 1), pl.ds(CH, CT)],
                    bufs[p].at[pl.ds(r, 1), :],
                    sems[p],
                ).wait()

        @pl.when(j == 0)
        def _():
            start_tails(0, 0)

        @pl.when((j + 1 < G) & (j % 2 == 0))
        def _():
            start_tails(j + 1, 1)

        @pl.when((j + 1 < G) & (j % 2 == 1))
        def _():
            start_tails(j + 1, 0)

        x = out_ref[...]
        m = jnp.max(x, axis=1, keepdims=True)
        lse = m + jnp.log(jnp.sum(jnp.exp(x - m), axis=1, keepdims=True))
        lse_ref[...] = lse

        @pl.when(j == 0)
        def _():
            s_ref[...] = jnp.zeros((1, 1), jnp.float32)

        def tail_contrib(tw):
            tout = out_ref[:, CH:C]
            tg2 = jnp.sum(tw, axis=1, keepdims=True)
            s_ref[...] += (jnp.sum(lse * tg2, keepdims=True).reshape(1, 1)
                           - jnp.sum(tw * tout, keepdims=True).reshape(1, 1))

        @pl.when(j % 2 == 0)
        def _():
            wait_tails(0)
            tail_contrib(t0[...])

        @pl.when(j % 2 == 1)
        def _():
            wait_tails(1)
            tail_contrib(t1[...])

    return pl.pallas_call(
        body,
        grid=grid,
        in_specs=[
            pl.BlockSpec((G, 1, BLK), lambda j: (0, 0, 0),
                         memory_space=pltpu.SMEM),
            pl.BlockSpec((BLK, C), lambda j: (j, 0)),
            pl.BlockSpec(memory_space=pl.ANY),
        ],
        out_specs=[
            pl.BlockSpec((BLK, 1), lambda j: (j, 0)),
            pl.BlockSpec((1, 1), lambda j: (0, 0)),
        ],
        out_shape=[
            jax.ShapeDtypeStruct((B, 1), jnp.float32),
            jax.ShapeDtypeStruct((1, 1), jnp.float32),
        ],
        scratch_shapes=[
            pltpu.VMEM((BLK, CT), jnp.float32),
            pltpu.VMEM((BLK, CT), jnp.float32),
            pltpu.SemaphoreType.DMA,
            pltpu.SemaphoreType.DMA,
        ],
    )(idx2, output, weights)


def _tc_combine(lse, g2part, t1part, s_tc, *, B):
    def body(lse_ref, g2_ref, t1_ref, s_ref, L_ref):
        g2 = jnp.sum(g2_ref[...], axis=1, keepdims=True)
        L_ref[...] = (
            jnp.sum(lse_ref[...] * g2, keepdims=True).reshape(1, 1)
            - jnp.sum(t1_ref[...], keepdims=True).reshape(1, 1)
            + s_ref[...]
        ) / B

    L = pl.pallas_call(
        body,
        out_shape=jax.ShapeDtypeStruct((1, 1), jnp.float32),
    )(lse, g2part, t1part, s_tc)
    return L[0, 0]


def kernel(output, targets, indices, weights):
    B, C = output.shape
    CH = C // 128 * 128
    bpw = B // _NW
    K = 16
    nchunk = bpw // K
    idx3 = indices.reshape(_NW, nchunk, K)
    idx2 = indices.reshape(B // 128, 1, 128)
    g2part, t1part = _sc_head_stats(
        output, idx3, weights, B=B, CH=CH, bpw=bpw, K=K, nchunk=nchunk)
    lse, s_tc = _tc_lse_tail(output, idx2, weights, B=B, C=C, CH=CH)
    return _tc_combine(lse, g2part, t1part, s_tc, B=B)
